# baseline (speedup 1.0000x reference)
import functools

import jax
import jax.numpy as jnp
from jax import lax
from jax.experimental import pallas as pl
from jax.experimental.pallas import tpu as pltpu
from jax.experimental.pallas import tpu_sc as plsc

_BB = 128
_NBUF = 2


@functools.cache
def _build(T: int, B: int, dim: int):
    mesh = plsc.VectorSubcoreMesh(core_axis_name="c", subcore_axis_name="s")
    num_workers = mesh.num_cores * mesh.num_subcores
    nc = mesh.num_cores
    chunks_total = T * (B // _BB)
    chunks_per_worker = chunks_total // num_workers
    b_chunks = B // _BB

    @functools.partial(
        pl.kernel,
        out_type=jax.ShapeDtypeStruct((T, dim, B), jnp.float32),
        mesh=mesh,
        scratch_types=[
            pltpu.VMEM((_BB,), jnp.int32),
            pltpu.VMEM((_BB,), jnp.int32),
            pltpu.VMEM((_BB,), jnp.int32),
            pltpu.VMEM((_BB,), jnp.int32),
            pltpu.VMEM((_BB, 2 * dim), jnp.float32),
            pltpu.VMEM((_BB, 2 * dim), jnp.float32),
            pltpu.VMEM((dim, _BB), jnp.float32),
            pltpu.VMEM((dim, _BB), jnp.float32),
            pltpu.SemaphoreType.DMA,
            pltpu.SemaphoreType.DMA,
            pltpu.SemaphoreType.DMA,
            pltpu.SemaphoreType.DMA,
        ],
        compiler_params=pltpu.CompilerParams(
            use_tc_tiling_on_sc=True, needs_layout_passes=False),
    )
    def body(tok_hbm, tab_hbm, out_hbm,
             tok0, tok1, idx0, idx1, g0, g1, o0, o1, gs0, gs1, ss0, ss1):
        bufs = ((tok0, idx0, g0, o0, gs0, ss0), (tok1, idx1, g1, o1, gs1, ss1))
        wid = lax.axis_index("s") * nc + lax.axis_index("c")
        chunk_base = wid * chunks_per_worker
        iota16 = jax.lax.iota(jnp.int32, 16)

        @pl.loop(0, chunks_per_worker, step=_NBUF)
        def _outer(k0):
            descs = []
            for b in range(_NBUF):
                tok_v, idx_v, g_v, o_v, gsem, ssem = bufs[b]
                g = chunk_base + k0 + b
                t = g // b_chunks
                b0 = (g % b_chunks) * _BB

                @pl.when(k0 > 0)
                def _drain():
                    pltpu.make_async_copy(
                        o_v, out_hbm.at[0, :, pl.ds(0, _BB)], ssem).wait()

                pltpu.sync_copy(tok_hbm.at[t, pl.ds(b0, _BB)], tok_v)
                for j0 in range(0, _BB, 16):
                    idx_v[pl.ds(j0, 16)] = (
                        tok_v[pl.ds(j0, 16)] >> jnp.int32(1))
                descs.append(pltpu.async_copy(tab_hbm.at[idx_v], g_v, gsem))
            for b in range(_NBUF):
                tok_v, idx_v, g_v, o_v, gsem, ssem = bufs[b]
                g = chunk_base + k0 + b
                t = g // b_chunks
                b0 = (g % b_chunks) * _BB
                descs[b].wait()
                for j0 in range(0, _BB, 16):
                    rows = iota16 + jnp.int32(j0)
                    par = (tok_v[pl.ds(j0, 16)] & jnp.int32(1)) * jnp.int32(dim)
                    for c in range(dim):
                        o_v[c, pl.ds(j0, 16)] = plsc.load_gather(
                            g_v, [rows, par + jnp.int32(c)])
                pltpu.async_copy(o_v, out_hbm.at[t, :, pl.ds(b0, _BB)], ssem)

        for b in range(_NBUF):
            _, _, _, o_v, _, ssem = bufs[b]
            pltpu.make_async_copy(
                o_v, out_hbm.at[0, :, pl.ds(0, _BB)], ssem).wait()

    return body


def kernel(token_ids, weight):
    n_tokens, seq = token_ids.shape
    n_rows, dim = weight.shape
    tok_t = token_ids.T.astype(jnp.int32)
    tab = weight.reshape(n_rows // 2, 2 * dim)
    out = _build(seq, n_tokens, dim)(tok_t, tab)
    return out.transpose(2, 0, 1)


# speedup vs baseline: 1.4479x; 1.4479x over previous
import functools

import jax
import jax.numpy as jnp
from jax import lax
from jax.experimental import pallas as pl
from jax.experimental.pallas import tpu as pltpu
from jax.experimental.pallas import tpu_sc as plsc

_BB = 128
_NBUF = 2


@functools.cache
def _build(T: int, B: int, dim: int):
    mesh = plsc.VectorSubcoreMesh(core_axis_name="c", subcore_axis_name="s")
    num_workers = mesh.num_cores * mesh.num_subcores
    nc = mesh.num_cores
    chunks_total = T * (B // _BB)
    chunks_per_worker = chunks_total // num_workers
    b_chunks = B // _BB

    @functools.partial(
        pl.kernel,
        out_type=jax.ShapeDtypeStruct((T, dim, B), jnp.float32),
        mesh=mesh,
        scratch_types=[
            pltpu.VMEM((_BB,), jnp.int32),
            pltpu.VMEM((_BB,), jnp.int32),
            pltpu.VMEM((_BB,), jnp.int32),
            pltpu.VMEM((_BB,), jnp.int32),
            pltpu.VMEM((_BB, 2 * dim), jnp.float32),
            pltpu.VMEM((_BB, 2 * dim), jnp.float32),
            pltpu.VMEM((dim, _BB), jnp.float32),
            pltpu.VMEM((dim, _BB), jnp.float32),
            pltpu.SemaphoreType.DMA,
            pltpu.SemaphoreType.DMA,
            pltpu.SemaphoreType.DMA,
            pltpu.SemaphoreType.DMA,
        ],
        compiler_params=pltpu.CompilerParams(
            use_tc_tiling_on_sc=True, needs_layout_passes=False),
    )
    def body(tok_hbm, tab_hbm, out_hbm,
             tok0, tok1, idx0, idx1, g0, g1, o0, o1, gs0, gs1, ss0, ss1):
        bufs = ((tok0, idx0, g0, o0, gs0, ss0), (tok1, idx1, g1, o1, gs1, ss1))
        wid = lax.axis_index("s") * nc + lax.axis_index("c")
        chunk_base = wid * chunks_per_worker
        iota16 = jax.lax.iota(jnp.int32, 16)

        @pl.loop(0, chunks_per_worker, step=_NBUF)
        def _outer(k0):
            descs = []
            for b in range(_NBUF):
                tok_v, idx_v, g_v, o_v, gsem, ssem = bufs[b]
                g = chunk_base + k0 + b
                t = g // b_chunks
                b0 = (g % b_chunks) * _BB

                @pl.when(k0 > 0)
                def _drain():
                    pltpu.make_async_copy(
                        o_v, out_hbm.at[0, :, pl.ds(0, _BB)], ssem).wait()

                pltpu.sync_copy(tok_hbm.at[t, pl.ds(b0, _BB)], tok_v)
                for j0 in range(0, _BB, 16):
                    idx_v[pl.ds(j0, 16)] = (
                        tok_v[pl.ds(j0, 16)] >> jnp.int32(1))
                descs.append(pltpu.async_copy(tab_hbm.at[idx_v], g_v, gsem))
            for b in range(_NBUF):
                tok_v, idx_v, g_v, o_v, gsem, ssem = bufs[b]
                g = chunk_base + k0 + b
                t = g // b_chunks
                b0 = (g % b_chunks) * _BB
                descs[b].wait()
                for j0 in range(0, _BB, 16):
                    rows = iota16 + jnp.int32(j0)
                    par = (tok_v[pl.ds(j0, 16)] & jnp.int32(1)) * jnp.int32(dim)

                    @plsc.parallel_loop(0, dim, unroll=8)
                    def _c(c, rows=rows, par=par, j0=j0, g_v=g_v, o_v=o_v):
                        o_v[c, pl.ds(j0, 16)] = plsc.load_gather(
                            g_v, [rows, par + c])
                pltpu.async_copy(o_v, out_hbm.at[t, :, pl.ds(b0, _BB)], ssem)

        for b in range(_NBUF):
            _, _, _, o_v, _, ssem = bufs[b]
            pltpu.make_async_copy(
                o_v, out_hbm.at[0, :, pl.ds(0, _BB)], ssem).wait()

    return body


def kernel(token_ids, weight):
    n_tokens, seq = token_ids.shape
    n_rows, dim = weight.shape
    tok_t = token_ids.T.astype(jnp.int32)
    tab = weight.reshape(n_rows // 2, 2 * dim)
    out = _build(seq, n_tokens, dim)(tok_t, tab)
    return out.transpose(2, 0, 1)
